# baseline (speedup 1.0000x reference)
import functools

import jax
import jax.numpy as jnp
from jax import lax
from jax.experimental import pallas as pl
from jax.experimental.pallas import tpu as pltpu
from jax.experimental.pallas import tpu_sc as plsc

B, L, V, D = 4096, 26, 1000, 16
NC, NS, LANES = 2, 16, 16
NW = NC * NS
BPW = B // NW
NG = BPW // LANES


def _gather_sum(x3, table_flat):
    mesh = plsc.VectorSubcoreMesh(core_axis_name="c", subcore_axis_name="s")

    @functools.partial(
        pl.kernel,
        out_type=jax.ShapeDtypeStruct((NW, D, BPW), jnp.float32),
        mesh=mesh,
        scratch_types=[
            pltpu.VMEM((V * D,), jnp.float32),
            pltpu.VMEM((BPW * L,), jnp.int32),
            pltpu.VMEM((D, BPW), jnp.float32),
        ],
        compiler_params=pltpu.CompilerParams(needs_layout_passes=False),
    )
    def body(x_hbm, table_hbm, out_hbm, table_v, idx_v, s_v):
        wid = lax.axis_index("s") * NC + lax.axis_index("c")
        pltpu.sync_copy(table_hbm, table_v)
        pltpu.sync_copy(x_hbm.at[wid], idx_v)
        lanes = jax.lax.iota(jnp.int32, LANES)

        zero = jnp.zeros((LANES,), jnp.float32)
        for g in range(NG):
            for d in range(D):
                s_v[d, pl.ds(g * LANES, LANES)] = zero

        def step(l, carry):
            for g in range(NG):
                col = g * LANES
                base = plsc.load_gather(idx_v, [(col + lanes) * L + l]) * D
                for d in range(D):
                    plsc.addupdate(
                        s_v.at[d, pl.ds(col, LANES)],
                        plsc.load_gather(table_v, [base + d]),
                    )
            return carry

        lax.fori_loop(0, L, step, 0)
        pltpu.sync_copy(s_v, out_hbm.at[wid])

    return body(x3, table_flat)


VBLK = 200


def _decode(s2, W_dec):
    def mm(s_ref, w_ref, o_ref):
        o_ref[...] = lax.dot_general(
            w_ref[...].astype(jnp.bfloat16), s_ref[...].astype(jnp.bfloat16),
            dimension_numbers=(((1,), (0,)), ((), ())),
            preferred_element_type=jnp.float32,
        )

    return pl.pallas_call(
        mm,
        grid=(V // VBLK,),
        in_specs=[
            pl.BlockSpec((D, B), lambda i: (0, 0)),
            pl.BlockSpec((VBLK, D), lambda i: (i, 0)),
        ],
        out_specs=pl.BlockSpec((VBLK, B), lambda i: (i, 0)),
        out_shape=jax.ShapeDtypeStruct((V, B), jnp.float32),
    )(s2, W_dec)


def kernel(x, table, W_dec):
    x3 = x.astype(jnp.int32).reshape(NW, BPW * L)
    sT = _gather_sum(x3, table.reshape(V * D))
    s2 = sT.transpose(1, 0, 2).reshape(D, B)
    return _decode(s2, W_dec).T


# speedup vs baseline: 1.4460x; 1.4460x over previous
import functools

import jax
import jax.numpy as jnp
from jax import lax
from jax.experimental import pallas as pl
from jax.experimental.pallas import tpu as pltpu
from jax.experimental.pallas import tpu_sc as plsc

B, L, V, D = 4096, 26, 1000, 16
NC, NS, LANES = 2, 16, 16
NW = NC * NS
BPW = B // NW
NG = BPW // LANES


def _gather_sum(x3, table_flat):
    mesh = plsc.VectorSubcoreMesh(core_axis_name="c", subcore_axis_name="s")

    @functools.partial(
        pl.kernel,
        out_type=jax.ShapeDtypeStruct((NW, D, BPW), jnp.float32),
        mesh=mesh,
        scratch_types=[
            pltpu.VMEM((V * D,), jnp.float32),
            pltpu.VMEM((BPW * L,), jnp.int32),
            pltpu.VMEM((D, BPW), jnp.float32),
        ],
        compiler_params=pltpu.CompilerParams(needs_layout_passes=False),
    )
    def body(x_hbm, table_hbm, out_hbm, table_v, idx_v, s_v):
        wid = lax.axis_index("s") * NC + lax.axis_index("c")
        pltpu.sync_copy(table_hbm, table_v)
        pltpu.sync_copy(x_hbm.at[wid], idx_v)
        lanes = jax.lax.iota(jnp.int32, LANES)

        def group(g, carry):
            col = g * LANES
            row_base = (col + lanes) * L

            def step(i, accs):
                l = i * 2
                base0 = plsc.load_gather(idx_v, [row_base + l]) * D
                base1 = plsc.load_gather(idx_v, [row_base + l + 1]) * D
                return tuple(
                    accs[d]
                    + plsc.load_gather(table_v, [base0 + d])
                    + plsc.load_gather(table_v, [base1 + d])
                    for d in range(D)
                )

            zeros = tuple(jnp.zeros((LANES,), jnp.float32) for _ in range(D))
            accs = lax.fori_loop(0, L // 2, step, zeros)
            for d in range(D):
                s_v[d, pl.ds(col, LANES)] = accs[d]
            return carry

        lax.fori_loop(0, NG, group, 0)
        pltpu.sync_copy(s_v, out_hbm.at[wid])

    return body(x3, table_flat)


VBLK = 200


def _decode(s2, W_dec):
    def mm(s_ref, w_ref, o_ref):
        o_ref[...] = lax.dot_general(
            w_ref[...].astype(jnp.bfloat16), s_ref[...].astype(jnp.bfloat16),
            dimension_numbers=(((1,), (0,)), ((), ())),
            preferred_element_type=jnp.float32,
        )

    return pl.pallas_call(
        mm,
        grid=(V // VBLK,),
        in_specs=[
            pl.BlockSpec((D, B), lambda i: (0, 0)),
            pl.BlockSpec((VBLK, D), lambda i: (i, 0)),
        ],
        out_specs=pl.BlockSpec((VBLK, B), lambda i: (i, 0)),
        out_shape=jax.ShapeDtypeStruct((V, B), jnp.float32),
    )(s2, W_dec)


def kernel(x, table, W_dec):
    x3 = x.astype(jnp.int32).reshape(NW, BPW * L)
    sT = _gather_sum(x3, table.reshape(V * D))
    s2 = sT.transpose(1, 0, 2).reshape(D, B)
    return _decode(s2, W_dec).T


# speedup vs baseline: 1.6887x; 1.1679x over previous
import functools

import jax
import jax.numpy as jnp
from jax import lax
from jax.experimental import pallas as pl
from jax.experimental.pallas import tpu as pltpu
from jax.experimental.pallas import tpu_sc as plsc

B, L, V, D = 4096, 26, 1000, 16
NC, NS, LANES = 2, 16, 16
NW = NC * NS
BPW = B // NW
NG = BPW // LANES
TS = D + 1


def _gather_sum(x3, table_flat):
    mesh = plsc.VectorSubcoreMesh(core_axis_name="c", subcore_axis_name="s")

    @functools.partial(
        pl.kernel,
        out_type=jax.ShapeDtypeStruct((NW, D, BPW), jnp.float32),
        mesh=mesh,
        scratch_types=[
            pltpu.VMEM((V * TS,), jnp.float32),
            pltpu.VMEM((BPW * L,), jnp.int32),
            pltpu.VMEM((D, BPW), jnp.float32),
        ],
        compiler_params=pltpu.CompilerParams(needs_layout_passes=False),
    )
    def body(x_hbm, table_hbm, out_hbm, table_v, idx_v, s_v):
        wid = lax.axis_index("s") * NC + lax.axis_index("c")
        pltpu.sync_copy(table_hbm, table_v)
        pltpu.sync_copy(x_hbm.at[wid], idx_v)
        lanes = jax.lax.iota(jnp.int32, LANES)

        def group(g, carry):
            col = g * LANES
            row_base = (col + lanes) * L

            def step(i, accs):
                l = i * 2
                base0 = plsc.load_gather(idx_v, [row_base + l]) * TS
                base1 = plsc.load_gather(idx_v, [row_base + l + 1]) * TS
                return tuple(
                    accs[d]
                    + plsc.load_gather(table_v, [base0 + d])
                    + plsc.load_gather(table_v, [base1 + d])
                    for d in range(D)
                )

            zeros = tuple(jnp.zeros((LANES,), jnp.float32) for _ in range(D))
            accs = lax.fori_loop(0, L // 2, step, zeros)
            for d in range(D):
                s_v[d, pl.ds(col, LANES)] = accs[d]
            return carry

        lax.fori_loop(0, NG, group, 0)
        pltpu.sync_copy(s_v, out_hbm.at[wid])

    return body(x3, table_flat)


VBLK = 200


def _decode(s2, W_dec):
    def mm(s_ref, w_ref, o_ref):
        o_ref[...] = lax.dot_general(
            w_ref[...].astype(jnp.bfloat16), s_ref[...].astype(jnp.bfloat16),
            dimension_numbers=(((1,), (0,)), ((), ())),
            preferred_element_type=jnp.float32,
        )

    return pl.pallas_call(
        mm,
        grid=(V // VBLK,),
        in_specs=[
            pl.BlockSpec((D, B), lambda i: (0, 0)),
            pl.BlockSpec((VBLK, D), lambda i: (i, 0)),
        ],
        out_specs=pl.BlockSpec((VBLK, B), lambda i: (i, 0)),
        out_shape=jax.ShapeDtypeStruct((V, B), jnp.float32),
    )(s2, W_dec)


def kernel(x, table, W_dec):
    x3 = x.astype(jnp.int32).reshape(NW, BPW * L)
    table_pad = jnp.pad(table, ((0, 0), (0, TS - D))).reshape(V * TS)
    sT = _gather_sum(x3, table_pad)
    s2 = sT.transpose(1, 0, 2).reshape(D, B)
    return _decode(s2, W_dec).T


# speedup vs baseline: 1.7678x; 1.0468x over previous
import functools

import jax
import jax.numpy as jnp
from jax import lax
from jax.experimental import pallas as pl
from jax.experimental.pallas import tpu as pltpu
from jax.experimental.pallas import tpu_sc as plsc

B, L, V, D = 4096, 26, 1000, 16
NC, NS, LANES = 2, 16, 16
NW = NC * NS
BPW = B // NW
NG = BPW // LANES
TS = D + 1


def _gather_sum(x3, table_flat):
    mesh = plsc.VectorSubcoreMesh(core_axis_name="c", subcore_axis_name="s")

    @functools.partial(
        pl.kernel,
        out_type=jax.ShapeDtypeStruct((D, B), jnp.float32),
        mesh=mesh,
        scratch_types=[
            pltpu.VMEM((V * TS,), jnp.float32),
            pltpu.VMEM((BPW * L,), jnp.int32),
            pltpu.VMEM((D, BPW), jnp.float32),
        ],
        compiler_params=pltpu.CompilerParams(needs_layout_passes=False),
    )
    def body(x_hbm, table_hbm, out_hbm, table_v, idx_v, s_v):
        wid = lax.axis_index("s") * NC + lax.axis_index("c")
        pltpu.sync_copy(table_hbm, table_v)
        pltpu.sync_copy(x_hbm.at[wid], idx_v)
        lanes = jax.lax.iota(jnp.int32, LANES)

        def group(g, carry):
            col = g * LANES
            row_base = (col + lanes) * L

            def step(i, accs):
                l = i * 2
                base0 = plsc.load_gather(idx_v, [row_base + l]) * TS
                base1 = plsc.load_gather(idx_v, [row_base + l + 1]) * TS
                return tuple(
                    accs[d]
                    + plsc.load_gather(table_v, [base0 + d])
                    + plsc.load_gather(table_v, [base1 + d])
                    for d in range(D)
                )

            zeros = tuple(jnp.zeros((LANES,), jnp.float32) for _ in range(D))
            accs = lax.fori_loop(0, L // 2, step, zeros)
            for d in range(D):
                s_v[d, pl.ds(col, LANES)] = accs[d]
            return carry

        lax.fori_loop(0, NG, group, 0)
        pltpu.sync_copy(s_v, out_hbm.at[:, pl.ds(wid * BPW, BPW)])

    return body(x3, table_flat)


VBLK = 200


def _decode(s2, W_dec):
    def mm(s_ref, w_ref, o_ref):
        o_ref[...] = lax.dot_general(
            w_ref[...].astype(jnp.bfloat16), s_ref[...].astype(jnp.bfloat16),
            dimension_numbers=(((1,), (0,)), ((), ())),
            preferred_element_type=jnp.float32,
        )

    return pl.pallas_call(
        mm,
        grid=(V // VBLK,),
        in_specs=[
            pl.BlockSpec((D, B), lambda i: (0, 0)),
            pl.BlockSpec((VBLK, D), lambda i: (i, 0)),
        ],
        out_specs=pl.BlockSpec((VBLK, B), lambda i: (i, 0)),
        out_shape=jax.ShapeDtypeStruct((V, B), jnp.float32),
    )(s2, W_dec)


def kernel(x, table, W_dec):
    x3 = x.astype(jnp.int32).reshape(NW, BPW * L)
    table_pad = jnp.pad(table, ((0, 0), (0, TS - D))).reshape(V * TS)
    s2 = _gather_sum(x3, table_pad)
    return _decode(s2, W_dec).T


# speedup vs baseline: 1.7739x; 1.0034x over previous
import functools

import jax
import jax.numpy as jnp
from jax import lax
from jax.experimental import pallas as pl
from jax.experimental.pallas import tpu as pltpu
from jax.experimental.pallas import tpu_sc as plsc

B, L, V, D = 4096, 26, 1000, 16
NC, NS, LANES = 2, 16, 16
NW = NC * NS
BPW = B // NW
NG = BPW // LANES
TS = D + 1


def _gather_sum(x3, table_flat):
    mesh = plsc.VectorSubcoreMesh(core_axis_name="c", subcore_axis_name="s")

    @functools.partial(
        pl.kernel,
        out_type=jax.ShapeDtypeStruct((D, B), jnp.float32),
        mesh=mesh,
        scratch_types=[
            pltpu.VMEM((V * TS,), jnp.float32),
            pltpu.VMEM((BPW * L,), jnp.int32),
            pltpu.VMEM((D, BPW), jnp.float32),
        ],
        compiler_params=pltpu.CompilerParams(needs_layout_passes=False),
    )
    def body(x_hbm, table_hbm, out_hbm, table_v, idx_v, s_v):
        wid = lax.axis_index("s") * NC + lax.axis_index("c")
        pltpu.sync_copy(table_hbm, table_v)
        pltpu.sync_copy(x_hbm.at[wid], idx_v)
        lanes = jax.lax.iota(jnp.int32, LANES)

        def group(g, carry):
            col = g * LANES
            row_base = (col + lanes) * L

            def step(l, accs):
                base = plsc.load_gather(idx_v, [row_base + l]) * TS
                return tuple(
                    accs[d] + plsc.load_gather(table_v, [base + d])
                    for d in range(D)
                )

            zeros = tuple(jnp.zeros((LANES,), jnp.float32) for _ in range(D))
            accs = lax.fori_loop(0, L, step, zeros)
            for d in range(D):
                s_v[d, pl.ds(col, LANES)] = accs[d]
            return carry

        lax.fori_loop(0, NG, group, 0)
        pltpu.sync_copy(s_v, out_hbm.at[:, pl.ds(wid * BPW, BPW)])

    return body(x3, table_flat)


VBLK = 200


def _decode(s2, W_dec):
    def mm(s_ref, w_ref, o_ref):
        o_ref[...] = lax.dot_general(
            w_ref[...].astype(jnp.bfloat16), s_ref[...].astype(jnp.bfloat16),
            dimension_numbers=(((1,), (0,)), ((), ())),
            preferred_element_type=jnp.float32,
        )

    return pl.pallas_call(
        mm,
        grid=(V // VBLK,),
        in_specs=[
            pl.BlockSpec((D, B), lambda i: (0, 0)),
            pl.BlockSpec((VBLK, D), lambda i: (i, 0)),
        ],
        out_specs=pl.BlockSpec((VBLK, B), lambda i: (i, 0)),
        out_shape=jax.ShapeDtypeStruct((V, B), jnp.float32),
    )(s2, W_dec)


def kernel(x, table, W_dec):
    x3 = x.astype(jnp.int32).reshape(NW, BPW * L)
    table_pad = jnp.pad(table, ((0, 0), (0, TS - D))).reshape(V * TS)
    s2 = _gather_sum(x3, table_pad)
    return _decode(s2, W_dec).T


# speedup vs baseline: 1.7992x; 1.0143x over previous
import functools

import jax
import jax.numpy as jnp
from jax import lax
from jax.experimental import pallas as pl
from jax.experimental.pallas import tpu as pltpu
from jax.experimental.pallas import tpu_sc as plsc

B, L, V, D = 4096, 26, 1000, 16
NC, NS, LANES = 2, 16, 16
NW = NC * NS
BPW = B // NW
NG = BPW // LANES
TS = D + 1
DW = D // 2
TSW = DW + 1


def _gather_sum(x3, table_flat):
    mesh = plsc.VectorSubcoreMesh(core_axis_name="c", subcore_axis_name="s")

    @functools.partial(
        pl.kernel,
        out_type=jax.ShapeDtypeStruct((D, B), jnp.float32),
        mesh=mesh,
        scratch_types=[
            pltpu.VMEM((V * TSW,), jnp.int32),
            pltpu.VMEM((BPW * L,), jnp.int32),
            pltpu.VMEM((D, BPW), jnp.float32),
        ],
        compiler_params=pltpu.CompilerParams(needs_layout_passes=False),
    )
    def body(x_hbm, table_hbm, out_hbm, table_v, idx_v, s_v):
        wid = lax.axis_index("s") * NC + lax.axis_index("c")
        pltpu.sync_copy(table_hbm, table_v)
        pltpu.sync_copy(x_hbm.at[wid], idx_v)
        lanes = jax.lax.iota(jnp.int32, LANES)

        def group(g, carry):
            col = g * LANES
            row_base = (col + lanes) * L

            def step(l, accs):
                base = plsc.load_gather(idx_v, [row_base + l]) * TSW
                out = []
                for k in range(DW):
                    w = plsc.load_gather(table_v, [base + k])
                    lo = plsc.bitcast(w << 16, jnp.float32)
                    hi = plsc.bitcast(w & jnp.int32(-65536), jnp.float32)
                    out.append(accs[2 * k] + lo)
                    out.append(accs[2 * k + 1] + hi)
                return tuple(out)

            zeros = tuple(jnp.zeros((LANES,), jnp.float32) for _ in range(D))
            accs = lax.fori_loop(0, L, step, zeros)
            for d in range(D):
                s_v[d, pl.ds(col, LANES)] = accs[d]
            return carry

        lax.fori_loop(0, NG, group, 0)
        pltpu.sync_copy(s_v, out_hbm.at[:, pl.ds(wid * BPW, BPW)])

    return body(x3, table_flat)


VBLK = 200


def _decode(s2, W_dec):
    def mm(s_ref, w_ref, o_ref):
        o_ref[...] = lax.dot_general(
            w_ref[...].astype(jnp.bfloat16), s_ref[...].astype(jnp.bfloat16),
            dimension_numbers=(((1,), (0,)), ((), ())),
            preferred_element_type=jnp.float32,
        )

    return pl.pallas_call(
        mm,
        grid=(V // VBLK,),
        in_specs=[
            pl.BlockSpec((D, B), lambda i: (0, 0)),
            pl.BlockSpec((VBLK, D), lambda i: (i, 0)),
        ],
        out_specs=pl.BlockSpec((VBLK, B), lambda i: (i, 0)),
        out_shape=jax.ShapeDtypeStruct((V, B), jnp.float32),
    )(s2, W_dec)


def kernel(x, table, W_dec):
    x3 = x.astype(jnp.int32).reshape(NW, BPW * L)
    t16 = lax.bitcast_convert_type(table.astype(jnp.bfloat16), jnp.uint16)
    tw = (t16[:, 0::2].astype(jnp.uint32)
          | (t16[:, 1::2].astype(jnp.uint32) << 16)).astype(jnp.int32)
    tw_pad = jnp.pad(tw, ((0, 0), (0, TSW - DW))).reshape(V * TSW)
    s2 = _gather_sum(x3, tw_pad)
    return _decode(s2, W_dec).T


# speedup vs baseline: 1.8062x; 1.0039x over previous
import functools

import jax
import jax.numpy as jnp
from jax import lax
from jax.experimental import pallas as pl
from jax.experimental.pallas import tpu as pltpu
from jax.experimental.pallas import tpu_sc as plsc

B, L, V, D = 4096, 26, 1000, 16
NC, NS, LANES = 2, 16, 16
NW = NC * NS
BPW = B // NW
NG = BPW // LANES
TS = D + 1
DW = D // 2
TSW = DW + 1


def _gather_sum(x3, table_flat):
    mesh = plsc.VectorSubcoreMesh(core_axis_name="c", subcore_axis_name="s")

    @functools.partial(
        pl.kernel,
        out_type=jax.ShapeDtypeStruct((D, B), jnp.float32),
        mesh=mesh,
        scratch_types=[
            pltpu.VMEM((V * TSW,), jnp.int32),
            pltpu.VMEM((BPW * L,), jnp.int32),
            pltpu.VMEM((D, BPW), jnp.float32),
        ],
        compiler_params=pltpu.CompilerParams(needs_layout_passes=False),
    )
    def body(x_hbm, table_hbm, out_hbm, table_v, idx_v, s_v):
        wid = lax.axis_index("s") * NC + lax.axis_index("c")
        pltpu.sync_copy(table_hbm, table_v)
        pltpu.sync_copy(x_hbm.at[wid], idx_v)
        lanes = jax.lax.iota(jnp.int32, LANES)

        @plsc.parallel_loop(0, NG, unroll=2)
        def group(g):
            col = g * LANES
            row_base = (col + lanes) * L

            def step(l, accs):
                base = plsc.load_gather(idx_v, [row_base + l]) * TSW
                out = []
                for k in range(DW):
                    w = plsc.load_gather(table_v, [base + k])
                    lo = plsc.bitcast(w << 16, jnp.float32)
                    hi = plsc.bitcast(w & jnp.int32(-65536), jnp.float32)
                    out.append(accs[2 * k] + lo)
                    out.append(accs[2 * k + 1] + hi)
                return tuple(out)

            zeros = tuple(jnp.zeros((LANES,), jnp.float32) for _ in range(D))
            accs = lax.fori_loop(0, L, step, zeros)
            for d in range(D):
                s_v[d, pl.ds(col, LANES)] = accs[d]
        pltpu.sync_copy(s_v, out_hbm.at[:, pl.ds(wid * BPW, BPW)])

    return body(x3, table_flat)


VBLK = 200


def _decode(s2, W_dec):
    def mm(s_ref, w_ref, o_ref):
        o_ref[...] = lax.dot_general(
            w_ref[...].astype(jnp.bfloat16), s_ref[...].astype(jnp.bfloat16),
            dimension_numbers=(((1,), (0,)), ((), ())),
            preferred_element_type=jnp.float32,
        )

    return pl.pallas_call(
        mm,
        grid=(V // VBLK,),
        in_specs=[
            pl.BlockSpec((D, B), lambda i: (0, 0)),
            pl.BlockSpec((VBLK, D), lambda i: (i, 0)),
        ],
        out_specs=pl.BlockSpec((VBLK, B), lambda i: (i, 0)),
        out_shape=jax.ShapeDtypeStruct((V, B), jnp.float32),
    )(s2, W_dec)


def kernel(x, table, W_dec):
    x3 = x.astype(jnp.int32).reshape(NW, BPW * L)
    t16 = lax.bitcast_convert_type(table.astype(jnp.bfloat16), jnp.uint16)
    tw = (t16[:, 0::2].astype(jnp.uint32)
          | (t16[:, 1::2].astype(jnp.uint32) << 16)).astype(jnp.int32)
    tw_pad = jnp.pad(tw, ((0, 0), (0, TSW - DW))).reshape(V * TSW)
    s2 = _gather_sum(x3, tw_pad)
    return _decode(s2, W_dec).T
